# baseline (speedup 1.0000x reference)
import numpy as np
import jax
import jax.numpy as jnp
from jax.experimental import pallas as pl
from jax.experimental.pallas import tpu as pltpu

_B = 16
_P = 64
_D = 128
_M = 100000
_K = 9
_OUT = 224
_N = _B * _P
_BM = 2048
_MP = 102400
_NBLK = _MP // _BM
_BIG_I = 2 ** 30


def _bilinear_matrix(n_out: int, n_in: int) -> np.ndarray:
    i = np.arange(n_out)
    src = (i + 0.5) * (n_in / n_out) - 0.5
    j0 = np.floor(src).astype(np.int64)
    frac = (src - j0).astype(np.float32)
    a = np.zeros((n_out, n_in), np.float32)
    a[i, np.clip(j0, 0, n_in - 1)] += 1.0 - frac
    a[i, np.clip(j0 + 1, 0, n_in - 1)] += frac
    return a


_A_NP = _bilinear_matrix(_OUT, 8)


_BN = 256
_NB = _N // _BN


def _nn_body(emb_ref, bank_ref, score_ref, idx_ref, run_min, run_idx):
    m_i = pl.program_id(1)

    @pl.when(m_i == 0)
    def _init():
        run_min[...] = jnp.full((_BN, 1), jnp.inf, jnp.float32)
        run_idx[...] = jnp.zeros((_BN, 1), jnp.int32)

    x = emb_ref[...]
    y = bank_ref[...]
    xy = jax.lax.dot_general(x, y, (((1,), (1,)), ((), ())),
                             preferred_element_type=jnp.float32)
    y2 = jax.lax.dot_general(jnp.ones((1, _D), jnp.float32), y * y,
                             (((1,), (1,)), ((), ())),
                             preferred_element_type=jnp.float32)
    t = y2 - 2.0 * xy
    col = jax.lax.broadcasted_iota(jnp.int32, t.shape, 1) + m_i * _BM
    t = jnp.where(col < _M, t, jnp.inf)
    m = jnp.min(t, axis=1, keepdims=True)
    cand = jnp.where(t == m, col, _BIG_I)
    bi = jnp.min(cand, axis=1, keepdims=True)
    better = m < run_min[...]
    run_idx[...] = jnp.where(better, bi, run_idx[...])
    run_min[...] = jnp.where(better, m, run_min[...])

    @pl.when(m_i == _NBLK - 1)
    def _done():
        x2 = jnp.sum(x * x, axis=1, keepdims=True)
        score_ref[...] = jnp.sqrt(jnp.maximum(run_min[...] + x2, 0.0))
        idx_ref[...] = run_idx[...]


def _select_body(scores_ref, locs_ref, emb_ref, score_ref, nn_ref, feat_ref):
    s = scores_ref[...]
    m = jnp.max(s, axis=1, keepdims=True)
    colp = jax.lax.broadcasted_iota(jnp.int32, s.shape, 1)
    cand = jnp.where(s == m, colp, _BIG_I)
    mp = jnp.min(cand, axis=1, keepdims=True)
    nn = jnp.sum(jnp.where(colp == mp, locs_ref[...], 0), axis=1, keepdims=True)
    score_ref[...] = m
    nn_ref[...] = nn
    c = jax.lax.broadcasted_iota(jnp.int32, (_B, _N), 1)
    b = jax.lax.broadcasted_iota(jnp.int32, (_B, _N), 0)
    onehot = (c == b * _P + mp).astype(jnp.float32)
    feat_ref[...] = jax.lax.dot_general(
        onehot, emb_ref[...], (((1,), (0,)), ((), ())),
        preferred_element_type=jnp.float32)


def _gather_body(idx_ref, bank_ref, out_ref):
    del idx_ref
    out_ref[...] = bank_ref[...]


def _gather_rows(bank3d, idx):
    n = idx.shape[0]
    out = pl.pallas_call(
        _gather_body,
        grid_spec=pltpu.PrefetchScalarGridSpec(
            num_scalar_prefetch=1,
            grid=(n,),
            in_specs=[pl.BlockSpec((1, 1, _D), lambda g, idx_ref: (idx_ref[g], 0, 0))],
            out_specs=pl.BlockSpec((1, 1, _D), lambda g, idx_ref: (g, 0, 0)),
        ),
        out_shape=jax.ShapeDtypeStruct((n, 1, _D), jnp.float32),
    )(idx, bank3d)
    return out.reshape(n, _D)


def _dist_body(nns_ref, bank_ref, dist_ref):
    i = pl.program_id(0)
    x = nns_ref[...]
    y = bank_ref[...]
    xy = jax.lax.dot_general(x, y, (((1,), (1,)), ((), ())),
                             preferred_element_type=jnp.float32)
    y2 = jax.lax.dot_general(jnp.ones((1, _D), jnp.float32), y * y,
                             (((1,), (1,)), ((), ())),
                             preferred_element_type=jnp.float32)
    t = y2 - 2.0 * xy
    col = jax.lax.broadcasted_iota(jnp.int32, t.shape, 1) + i * _BM
    dist_ref[...] = jnp.where(col < _M, t, jnp.inf)


def _topk_body(dist_ref, sidx_ref):
    d = dist_ref[...]
    colf = jax.lax.broadcasted_iota(jnp.int32, d.shape, 1)
    for k in range(_K):
        m = jnp.min(d, axis=1, keepdims=True)
        cand = jnp.where(d == m, colf, _BIG_I)
        idx = jnp.min(cand, axis=1, keepdims=True)
        sidx_ref[:, k:k + 1] = idx
        d = jnp.where(colf == idx, jnp.inf, d)
    sidx_ref[:, _K:] = jnp.zeros((_B, 16 - _K), jnp.int32)


def _score_body(mf_ref, sf_ref, sc_ref, pred_ref):
    mf = mf_ref[...]
    sf = sf_ref[...]
    g = jax.lax.dot_general(mf, sf, (((1,), (1,)), ((), ())),
                            preferred_element_type=jnp.float32)
    mf2 = jnp.sum(mf * mf, axis=1, keepdims=True)
    sf2 = jax.lax.dot_general(jnp.ones((1, _D), jnp.float32), sf * sf,
                              (((1,), (1,)), ((), ())),
                              preferred_element_type=jnp.float32)
    d = jnp.sqrt(jnp.maximum(mf2 + sf2 - 2.0 * g, 0.0))
    col = jax.lax.broadcasted_iota(jnp.int32, d.shape, 1)
    row = jax.lax.broadcasted_iota(jnp.int32, d.shape, 0)
    valid = (col >= row * _K) & (col < (row + 1) * _K)
    mrow = jnp.max(jnp.where(valid, d, -jnp.inf), axis=1, keepdims=True)
    e = jnp.where(valid, jnp.exp(d - mrow), 0.0)
    ssum = jnp.sum(e, axis=1, keepdims=True)
    e0 = jnp.sum(jnp.where(col == row * _K, e, 0.0), axis=1, keepdims=True)
    w = 1.0 - e0 / ssum
    pred_ref[...] = w * sc_ref[...]


def _resize_body(s_ref, a_ref, at_ref, out_ref):
    s = s_ref[0]
    p = jax.lax.dot_general(a_ref[...], s, (((1,), (0,)), ((), ())),
                            preferred_element_type=jnp.float32)
    out_ref[0] = jax.lax.dot_general(p, at_ref[...], (((1,), (0,)), ((), ())),
                                     preferred_element_type=jnp.float32)


def kernel(embedding, memory_bank):
    emb = embedding.astype(jnp.float32)
    bank = jnp.pad(memory_bank.astype(jnp.float32), ((0, _MP - _M), (0, 0)))

    score_flat, loc_flat = pl.pallas_call(
        _nn_body,
        grid=(_NB, _NBLK),
        in_specs=[pl.BlockSpec((_BN, _D), lambda n, m: (n, 0)),
                  pl.BlockSpec((_BM, _D), lambda n, m: (m, 0))],
        out_specs=[pl.BlockSpec((_BN, 1), lambda n, m: (n, 0)),
                   pl.BlockSpec((_BN, 1), lambda n, m: (n, 0))],
        out_shape=[jax.ShapeDtypeStruct((_N, 1), jnp.float32),
                   jax.ShapeDtypeStruct((_N, 1), jnp.int32)],
        scratch_shapes=[pltpu.VMEM((_BN, 1), jnp.float32),
                        pltpu.VMEM((_BN, 1), jnp.int32)],
        compiler_params=pltpu.CompilerParams(
            dimension_semantics=("arbitrary", "arbitrary")),
    )(emb, bank)
    scores = score_flat.reshape(_B, _P)
    locs = loc_flat.reshape(_B, _P)

    score16, nn_idx, max_feat = pl.pallas_call(
        _select_body,
        out_shape=[jax.ShapeDtypeStruct((_B, 1), jnp.float32),
                   jax.ShapeDtypeStruct((_B, 1), jnp.int32),
                   jax.ShapeDtypeStruct((_B, _D), jnp.float32)],
    )(scores, locs, emb)

    bank3d = bank.reshape(_MP, 1, _D)
    nns = _gather_rows(bank3d, nn_idx.reshape(_B))

    dist = pl.pallas_call(
        _dist_body,
        grid=(_NBLK,),
        in_specs=[pl.BlockSpec((_B, _D), lambda i: (0, 0)),
                  pl.BlockSpec((_BM, _D), lambda i: (i, 0))],
        out_specs=pl.BlockSpec((_B, _BM), lambda i: (0, i)),
        out_shape=jax.ShapeDtypeStruct((_B, _MP), jnp.float32),
    )(nns, bank)

    sidx = pl.pallas_call(
        _topk_body,
        out_shape=jax.ShapeDtypeStruct((_B, 16), jnp.int32),
    )(dist)

    sup_feats = _gather_rows(bank3d, sidx[:, :_K].reshape(_B * _K))

    pred = pl.pallas_call(
        _score_body,
        out_shape=jax.ShapeDtypeStruct((_B, 1), jnp.float32),
    )(max_feat, sup_feats, score16)

    a_mat = jnp.asarray(_A_NP)
    maps = pl.pallas_call(
        _resize_body,
        grid=(_B,),
        in_specs=[pl.BlockSpec((1, 8, 8), lambda b: (b, 0, 0)),
                  pl.BlockSpec((_OUT, 8), lambda b: (0, 0)),
                  pl.BlockSpec((8, _OUT), lambda b: (0, 0))],
        out_specs=pl.BlockSpec((1, _OUT, _OUT), lambda b: (b, 0, 0)),
        out_shape=jax.ShapeDtypeStruct((_B, _OUT, _OUT), jnp.float32),
    )(scores.reshape(_B, 8, 8), a_mat, a_mat.T)

    return maps.reshape(_B, 1, _OUT, _OUT), pred.reshape(_B)
